# baseline (speedup 1.0000x reference)
import functools

import numpy as np

import jax
import jax.numpy as jnp
from jax import lax
from jax.experimental import pallas as pl
from jax.experimental.pallas import tpu as pltpu
from jax.experimental.pallas import tpu_sc as plsc

P = 7
RSCALE = 1.2
BSCALE = 1.4
NUM_SC = 2
NUM_SUBCORES = 16
NUM_WORKERS = NUM_SC * NUM_SUBCORES


def _integral_body(x_ref, u_ref, bd_ref, rot_ref, out_ref):
    x = x_ref[0, 0]
    s1 = lax.dot_general(x, u_ref[...], (((1,), (0,)), ((), ())),
                         preferred_element_type=jnp.float32,
                         precision=lax.Precision.HIGHEST)
    s = lax.dot_general(bd_ref[...], s1, (((1,), (0,)), ((), ())),
                        preferred_element_type=jnp.float32,
                        precision=lax.Precision.HIGHEST)
    d = s.shape[0] // 56
    s3 = s.reshape(d, 56, 128).reshape(d, 56 * 128)
    out_ref[...] = lax.dot_general(s3, rot_ref[...], (((0,), (0,)), ((), ())),
                                   preferred_element_type=jnp.float32,
                                   precision=lax.Precision.HIGHEST)


def _bins_body(rois_ref, sscale_ref, idx_ref, sc_ref, h_size, w_size):
    r = rois_ref[...]
    s = sscale_ref[0, 0]
    bidx = r[:, 0:1].astype(jnp.int32)
    x1 = r[:, 1:2] * s
    y1 = r[:, 2:3] * s
    x2 = r[:, 3:4] * s
    y2 = r[:, 4:5] * s
    roi_w = jnp.maximum(x2 - x1, 0.1)
    roi_h = jnp.maximum(y2 - y1, 0.1)
    cx = 0.5 * (x1 + x2)
    cy = 0.5 * (y1 + y2)
    w_s = roi_w * RSCALE
    h_s = roi_h * RSCALE
    x1s = cx - 0.5 * w_s
    y1s = cy - 0.5 * h_s
    bin_w = w_s / P
    bin_h = h_s / P

    col = lax.broadcasted_iota(jnp.int32, (1, 256), 1)
    pc = jnp.minimum(col // 4, P * P - 1)
    corner = col % 4
    ph = (pc // P).astype(jnp.float32)
    pw = (pc % P).astype(jnp.float32)
    cyb = y1s + (ph + 0.5) * bin_h
    cxb = x1s + (pw + 0.5) * bin_w
    half_h = 0.5 * BSCALE * bin_h
    half_w = 0.5 * BSCALE * bin_w
    hs = jnp.clip(jnp.floor(cyb - half_h), 0.0, float(h_size))
    he = jnp.clip(jnp.ceil(cyb + half_h), 0.0, float(h_size))
    ws = jnp.clip(jnp.floor(cxb - half_w), 0.0, float(w_size))
    we = jnp.clip(jnp.ceil(cxb + half_w), 0.0, float(w_size))
    ii = jnp.where(corner % 2 == 0, he, hs).astype(jnp.int32)
    jj = jnp.where(corner < 2, we, ws).astype(jnp.int32)
    slab = 56 * 128
    base = (bidx * (P * P) + pc) * slab
    idx_ref[...] = jnp.where(col < P * P * 4, base + ii * 128 + jj, 0)

    col2 = lax.broadcasted_iota(jnp.int32, (1, 56 * 16), 1)
    pc2 = jnp.minimum(col2 // 16, P * P - 1)
    ph2 = (pc2 // P).astype(jnp.float32)
    pw2 = (pc2 % P).astype(jnp.float32)
    cyb2 = y1s + (ph2 + 0.5) * bin_h
    cxb2 = x1s + (pw2 + 0.5) * bin_w
    hh2 = 0.5 * BSCALE * bin_h
    hw2 = 0.5 * BSCALE * bin_w
    hs2 = jnp.clip(jnp.floor(cyb2 - hh2), 0.0, float(h_size))
    he2 = jnp.clip(jnp.ceil(cyb2 + hh2), 0.0, float(h_size))
    ws2 = jnp.clip(jnp.floor(cxb2 - hw2), 0.0, float(w_size))
    we2 = jnp.clip(jnp.ceil(cxb2 + hw2), 0.0, float(w_size))
    cnt = (he2 - hs2) * (we2 - ws2)
    sc_ref[...] = jnp.where(cnt > 0.0, 1.0 / jnp.maximum(cnt, 1.0), 0.0)


def _make_sc_gather(n_bins, n_rows):
    bins_per_worker = n_bins // NUM_WORKERS
    rois_per_worker = bins_per_worker // (P * P)
    n_chunks = rois_per_worker * 2
    mesh = plsc.VectorSubcoreMesh(core_axis_name="c", subcore_axis_name="s")

    @functools.partial(
        pl.kernel, mesh=mesh,
        out_type=jax.ShapeDtypeStruct((n_bins, 16), jnp.float32),
        scratch_types=[
            pltpu.VMEM((n_chunks, 128), jnp.int32),
            pltpu.VMEM((128, 16), jnp.float32),
            pltpu.VMEM((128, 16), jnp.float32),
            pltpu.VMEM((rois_per_worker * 56, 16), jnp.float32),
            pltpu.VMEM((bins_per_worker, 16), jnp.float32),
            pltpu.SemaphoreType.DMA,
            pltpu.SemaphoreType.DMA,
        ],
        compiler_params=pltpu.CompilerParams(use_tc_tiling_on_sc=False),
    )
    def sc_gather(idx_hbm, scale_hbm, table_hbm, out_hbm,
                  idx_v, rows_a, rows_b, scale_v, out_v, sem_a, sem_b):
        wid = lax.axis_index("s") * NUM_SC + lax.axis_index("c")
        pltpu.sync_copy(idx_hbm.at[pl.ds(wid * n_chunks, n_chunks)], idx_v)
        pltpu.sync_copy(
            scale_hbm.at[pl.ds(wid * rois_per_worker * 56,
                               rois_per_worker * 56)], scale_v)

        bufs = (rows_a, rows_b)
        sems = (sem_a, sem_b)

        def start(k):
            return pltpu.async_copy(table_hbm.at[idx_v.at[k]],
                                    bufs[k % 2], sems[k % 2])

        pending = start(0)
        for k in range(n_chunks):
            nxt = start(k + 1) if k + 1 < n_chunks else None
            pending.wait()
            rows = bufs[k % 2]
            r_local = k // 2
            nb = 32 if k % 2 == 0 else P * P - 32
            for i in range(nb):
                b_local = (k % 2) * 32 + i
                acc = (rows[4 * i, :] - rows[4 * i + 1, :]
                       - rows[4 * i + 2, :] + rows[4 * i + 3, :])
                out_v[r_local * (P * P) + b_local, :] = (
                    acc * scale_v[r_local * 56 + b_local, :])
            pending = nxt
        pltpu.sync_copy(
            out_v, out_hbm.at[pl.ds(wid * bins_per_worker, bins_per_worker)])

    return sc_gather


def kernel(rois, features, stride):
    B, C, H, W = features.shape
    D = C // (P * P)
    N = rois.shape[0]
    n_bins = N * P * P
    slab = 56 * 128
    n_rows = B * P * P * slab

    x_t = (features.reshape(B, D, P * P, H, W)
           .transpose(0, 2, 1, 3, 4)
           .reshape(B, P * P, D * H, W))

    u_mat = (np.arange(W)[:, None] < np.arange(128)[None, :]).astype(np.float32)
    u_mat[:, W + 1:] = 0.0
    rr = np.arange(D * 56)
    cc = np.arange(D * H)
    bd_mat = ((rr[:, None] // 56 == cc[None, :] // H)
              & (cc[None, :] % H < rr[:, None] % 56)
              & (rr[:, None] % 56 <= H)).astype(np.float32)
    rot_mat = np.zeros((D, 16), np.float32)
    rot_mat[np.arange(D), np.arange(D)] = 1.0

    table = pl.pallas_call(
        _integral_body,
        grid=(B * P * P,),
        in_specs=[
            pl.BlockSpec((1, 1, D * H, W),
                         lambda g: (g // (P * P), g % (P * P), 0, 0)),
            pl.BlockSpec((W, 128), lambda g: (0, 0)),
            pl.BlockSpec((D * 56, D * H), lambda g: (0, 0)),
            pl.BlockSpec((D, 16), lambda g: (0, 0)),
        ],
        out_specs=pl.BlockSpec((slab, 16), lambda g: (g, 0)),
        out_shape=jax.ShapeDtypeStruct((n_rows, 16), jnp.float32),
    )(x_t, jnp.asarray(u_mat), jnp.asarray(bd_mat), jnp.asarray(rot_mat))

    sscale = (1.0 / stride) * jnp.ones((1, 1), jnp.float32)
    bins_body = functools.partial(_bins_body, h_size=H, w_size=W)
    idx4, scale16 = pl.pallas_call(
        bins_body,
        out_shape=[
            jax.ShapeDtypeStruct((N, 256), jnp.int32),
            jax.ShapeDtypeStruct((N, 56 * 16), jnp.float32),
        ],
    )(rois.astype(jnp.float32), sscale)

    idx_flat = idx4.reshape(N * 2, 128)
    scale16 = scale16.reshape(N * 56, 16)

    out_rows = _make_sc_gather(n_bins, n_rows)(idx_flat, scale16, table)

    return (out_rows[:, :D].reshape(N, P * P, D)
            .transpose(0, 2, 1).reshape(N, D, P, P))


# speedup vs baseline: 1.9073x; 1.9073x over previous
import functools

import numpy as np

import jax
import jax.numpy as jnp
from jax import lax
from jax.experimental import pallas as pl
from jax.experimental.pallas import tpu as pltpu
from jax.experimental.pallas import tpu_sc as plsc

P = 7
RSCALE = 1.2
BSCALE = 1.4
NUM_SC = 2
NUM_SUBCORES = 16
NUM_WORKERS = NUM_SC * NUM_SUBCORES


def _fused_tc_body(x_ref, u_ref, bd_ref, rois_ref, sscale_ref,
                   tab_ref, idx_ref, sc_ref, h_size, w_size):
    x = x_ref[0, 0]
    a = lax.dot_general(bd_ref[...], x, (((1,), (0,)), ((), ())),
                        preferred_element_type=jnp.float32)
    tab_ref[0] = lax.dot_general(a, u_ref[...], (((1,), (0,)), ((), ())),
                                 preferred_element_type=jnp.float32)

    @pl.when(pl.program_id(0) == 0)
    def _():
        _bins_body(rois_ref, sscale_ref, idx_ref, sc_ref, h_size, w_size)


def _bins_body(rois_ref, sscale_ref, idx_ref, sc_ref, h_size, w_size):
    r = rois_ref[...]
    s = sscale_ref[0, 0]
    bidx = r[:, 0:1].astype(jnp.int32)
    x1 = r[:, 1:2] * s
    y1 = r[:, 2:3] * s
    x2 = r[:, 3:4] * s
    y2 = r[:, 4:5] * s
    roi_w = jnp.maximum(x2 - x1, 0.1)
    roi_h = jnp.maximum(y2 - y1, 0.1)
    cx = 0.5 * (x1 + x2)
    cy = 0.5 * (y1 + y2)
    w_s = roi_w * RSCALE
    h_s = roi_h * RSCALE
    x1s = cx - 0.5 * w_s
    y1s = cy - 0.5 * h_s
    bin_w = w_s / P
    bin_h = h_s / P

    col = lax.broadcasted_iota(jnp.int32, (1, 256), 1)
    pc = jnp.minimum(col // 4, P * P - 1)
    corner = col % 4
    ph = (pc // P).astype(jnp.float32)
    pw = (pc % P).astype(jnp.float32)
    cyb = y1s + (ph + 0.5) * bin_h
    cxb = x1s + (pw + 0.5) * bin_w
    half_h = 0.5 * BSCALE * bin_h
    half_w = 0.5 * BSCALE * bin_w
    hs = jnp.clip(jnp.floor(cyb - half_h), 0.0, float(h_size))
    he = jnp.clip(jnp.ceil(cyb + half_h), 0.0, float(h_size))
    ws = jnp.clip(jnp.floor(cxb - half_w), 0.0, float(w_size))
    we = jnp.clip(jnp.ceil(cxb + half_w), 0.0, float(w_size))
    ii = jnp.where(corner % 2 == 0, he, hs).astype(jnp.int32)
    jj = jnp.where(corner < 2, we, ws).astype(jnp.int32)
    wj = w_size + 1
    slab = (h_size + 1) * wj
    base = (bidx * (P * P) + pc) * slab
    idx_ref[...] = jnp.where(col < P * P * 4, base + ii * wj + jj, 0)

    col2 = lax.broadcasted_iota(jnp.int32, (1, 56 * 16), 1)
    pc2 = jnp.minimum(col2 // 16, P * P - 1)
    ph2 = (pc2 // P).astype(jnp.float32)
    pw2 = (pc2 % P).astype(jnp.float32)
    cyb2 = y1s + (ph2 + 0.5) * bin_h
    cxb2 = x1s + (pw2 + 0.5) * bin_w
    hh2 = 0.5 * BSCALE * bin_h
    hw2 = 0.5 * BSCALE * bin_w
    hs2 = jnp.clip(jnp.floor(cyb2 - hh2), 0.0, float(h_size))
    he2 = jnp.clip(jnp.ceil(cyb2 + hh2), 0.0, float(h_size))
    ws2 = jnp.clip(jnp.floor(cxb2 - hw2), 0.0, float(w_size))
    we2 = jnp.clip(jnp.ceil(cxb2 + hw2), 0.0, float(w_size))
    cnt = (he2 - hs2) * (we2 - ws2)
    sc_ref[...] = jnp.where(cnt > 0.0, 1.0 / jnp.maximum(cnt, 1.0), 0.0)


def _make_sc_gather(n_bins, n_rows):
    bins_per_worker = n_bins // NUM_WORKERS
    rois_per_worker = bins_per_worker // (P * P)
    n_chunks = rois_per_worker * 2
    mesh = plsc.VectorSubcoreMesh(core_axis_name="c", subcore_axis_name="s")

    @functools.partial(
        pl.kernel, mesh=mesh,
        out_type=jax.ShapeDtypeStruct((n_bins, 16), jnp.float32),
        scratch_types=[
            pltpu.VMEM((n_chunks, 128), jnp.int32),
            pltpu.VMEM((128, 16), jnp.float32),
            pltpu.VMEM((128, 16), jnp.float32),
            pltpu.VMEM((rois_per_worker * 56, 16), jnp.float32),
            pltpu.VMEM((bins_per_worker, 16), jnp.float32),
            pltpu.SemaphoreType.DMA,
            pltpu.SemaphoreType.DMA,
        ],
        compiler_params=pltpu.CompilerParams(use_tc_tiling_on_sc=False),
    )
    def sc_gather(idx_hbm, scale_hbm, table_hbm, out_hbm,
                  idx_v, rows_a, rows_b, scale_v, out_v, sem_a, sem_b):
        wid = lax.axis_index("s") * NUM_SC + lax.axis_index("c")
        pltpu.sync_copy(idx_hbm.at[pl.ds(wid * n_chunks, n_chunks)], idx_v)
        pltpu.sync_copy(
            scale_hbm.at[pl.ds(wid * rois_per_worker * 56,
                               rois_per_worker * 56)], scale_v)

        bufs = (rows_a, rows_b)
        sems = (sem_a, sem_b)

        def start(k):
            return pltpu.async_copy(table_hbm.at[idx_v.at[k]],
                                    bufs[k % 2], sems[k % 2])

        pending = start(0)
        for k in range(n_chunks):
            nxt = start(k + 1) if k + 1 < n_chunks else None
            pending.wait()
            rows = bufs[k % 2]
            r_local = k // 2
            nb = 32 if k % 2 == 0 else P * P - 32
            for i in range(nb):
                b_local = (k % 2) * 32 + i
                acc = (rows[4 * i, :] - rows[4 * i + 1, :]
                       - rows[4 * i + 2, :] + rows[4 * i + 3, :])
                out_v[r_local * (P * P) + b_local, :] = (
                    acc * scale_v[r_local * 56 + b_local, :])
            pending = nxt
        pltpu.sync_copy(
            out_v, out_hbm.at[pl.ds(wid * bins_per_worker, bins_per_worker)])

    return sc_gather


def kernel(rois, features, stride):
    B, C, H, W = features.shape
    D = C // (P * P)
    N = rois.shape[0]
    n_bins = N * P * P
    n_rows = B * P * P * (H + 1) * (W + 1)

    x_t = (features.reshape(B, D, P * P, H, W)
           .transpose(0, 2, 1, 3, 4)
           .reshape(B, P * P, D * H, W))

    u_mat = (np.arange(W)[:, None] < np.arange(W + 1)[None, :]).astype(np.float32)
    rr = np.arange(D * (H + 1))
    cc = np.arange(D * H)
    bd_mat = ((rr[:, None] // (H + 1) == cc[None, :] // H)
              & (cc[None, :] % H < rr[:, None] % (H + 1))).astype(np.float32)

    sscale = (1.0 / stride) * jnp.ones((1, 1), jnp.float32)
    body = functools.partial(_fused_tc_body, h_size=H, w_size=W)
    table_dmaj, idx4, scale16 = pl.pallas_call(
        body,
        grid=(B * P * P,),
        in_specs=[
            pl.BlockSpec((1, 1, D * H, W),
                         lambda g: (g // (P * P), g % (P * P), 0, 0)),
            pl.BlockSpec((W, W + 1), lambda g: (0, 0)),
            pl.BlockSpec((D * (H + 1), D * H), lambda g: (0, 0)),
            pl.BlockSpec((N, 5), lambda g: (0, 0)),
            pl.BlockSpec((1, 1), lambda g: (0, 0)),
        ],
        out_specs=[
            pl.BlockSpec((1, D * (H + 1), W + 1), lambda g: (g, 0, 0)),
            pl.BlockSpec((N, 256), lambda g: (0, 0)),
            pl.BlockSpec((N, 56 * 16), lambda g: (0, 0)),
        ],
        out_shape=[
            jax.ShapeDtypeStruct((B * P * P, D * (H + 1), W + 1), jnp.float32),
            jax.ShapeDtypeStruct((N, 256), jnp.int32),
            jax.ShapeDtypeStruct((N, 56 * 16), jnp.float32),
        ],
    )(x_t, jnp.asarray(u_mat), jnp.asarray(bd_mat),
      rois.astype(jnp.float32), sscale)

    table = table_dmaj.reshape(B * P * P, D, H + 1, W + 1).transpose(0, 2, 3, 1)
    table = jnp.pad(table, ((0, 0), (0, 0), (0, 0), (0, 16 - D)))
    table = table.reshape(n_rows, 16)

    idx_flat = idx4.reshape(N * 2, 128)
    scale16 = scale16.reshape(N * 56, 16)

    out_rows = _make_sc_gather(n_bins, n_rows)(idx_flat, scale16, table)

    return (out_rows[:, :D].reshape(N, P * P, D)
            .transpose(0, 2, 1).reshape(N, D, P, P))


# speedup vs baseline: 1.9288x; 1.0113x over previous
import functools

import numpy as np

import jax
import jax.numpy as jnp
from jax import lax
from jax.experimental import pallas as pl
from jax.experimental.pallas import tpu as pltpu
from jax.experimental.pallas import tpu_sc as plsc

P = 7
RSCALE = 1.2
BSCALE = 1.4
NUM_SC = 2
NUM_SUBCORES = 16
NUM_WORKERS = NUM_SC * NUM_SUBCORES


def _split_dot(m16, x, dn):
    hi = x.astype(jnp.bfloat16)
    lo = (x - hi.astype(jnp.float32)).astype(jnp.bfloat16)
    return (lax.dot_general(m16, hi, dn, preferred_element_type=jnp.float32)
            + lax.dot_general(m16, lo, dn, preferred_element_type=jnp.float32))


def _fused_tc_body(x_ref, u_ref, bd_ref, rois_ref, sscale_ref,
                   tab_ref, idx_ref, sc_ref, h_size, w_size, n_batch):
    dnl = (((1,), (0,)), ((), ()))
    for t in range(n_batch):
        a = _split_dot(bd_ref[...], x_ref[0, t], dnl)
        hi = a.astype(jnp.bfloat16)
        lo = (a - hi.astype(jnp.float32)).astype(jnp.bfloat16)
        tab_ref[t] = (
            lax.dot_general(hi, u_ref[...], dnl,
                            preferred_element_type=jnp.float32)
            + lax.dot_general(lo, u_ref[...], dnl,
                              preferred_element_type=jnp.float32))

    @pl.when(pl.program_id(0) == 0)
    def _():
        _bins_body(rois_ref, sscale_ref, idx_ref, sc_ref, h_size, w_size)


def _bins_body(rois_ref, sscale_ref, idx_ref, sc_ref, h_size, w_size):
    r = rois_ref[...]
    s = sscale_ref[0, 0]
    bidx = r[:, 0:1].astype(jnp.int32)
    x1 = r[:, 1:2] * s
    y1 = r[:, 2:3] * s
    x2 = r[:, 3:4] * s
    y2 = r[:, 4:5] * s
    roi_w = jnp.maximum(x2 - x1, 0.1)
    roi_h = jnp.maximum(y2 - y1, 0.1)
    cx = 0.5 * (x1 + x2)
    cy = 0.5 * (y1 + y2)
    w_s = roi_w * RSCALE
    h_s = roi_h * RSCALE
    x1s = cx - 0.5 * w_s
    y1s = cy - 0.5 * h_s
    bin_w = w_s / P
    bin_h = h_s / P

    col = lax.broadcasted_iota(jnp.int32, (1, 256), 1)
    pc = jnp.minimum(col // 4, P * P - 1)
    corner = col % 4
    ph = (pc // P).astype(jnp.float32)
    pw = (pc % P).astype(jnp.float32)
    cyb = y1s + (ph + 0.5) * bin_h
    cxb = x1s + (pw + 0.5) * bin_w
    half_h = 0.5 * BSCALE * bin_h
    half_w = 0.5 * BSCALE * bin_w
    hs = jnp.clip(jnp.floor(cyb - half_h), 0.0, float(h_size))
    he = jnp.clip(jnp.ceil(cyb + half_h), 0.0, float(h_size))
    ws = jnp.clip(jnp.floor(cxb - half_w), 0.0, float(w_size))
    we = jnp.clip(jnp.ceil(cxb + half_w), 0.0, float(w_size))
    ii = jnp.where(corner % 2 == 0, he, hs).astype(jnp.int32)
    jj = jnp.where(corner < 2, we, ws).astype(jnp.int32)
    wj = w_size + 1
    slab = (h_size + 1) * wj
    base = (bidx * (P * P) + pc) * slab
    idx_ref[...] = jnp.where(col < P * P * 4, base + ii * wj + jj, 0)

    col2 = lax.broadcasted_iota(jnp.int32, (1, 56 * 16), 1)
    pc2 = jnp.minimum(col2 // 16, P * P - 1)
    ph2 = (pc2 // P).astype(jnp.float32)
    pw2 = (pc2 % P).astype(jnp.float32)
    cyb2 = y1s + (ph2 + 0.5) * bin_h
    cxb2 = x1s + (pw2 + 0.5) * bin_w
    hh2 = 0.5 * BSCALE * bin_h
    hw2 = 0.5 * BSCALE * bin_w
    hs2 = jnp.clip(jnp.floor(cyb2 - hh2), 0.0, float(h_size))
    he2 = jnp.clip(jnp.ceil(cyb2 + hh2), 0.0, float(h_size))
    ws2 = jnp.clip(jnp.floor(cxb2 - hw2), 0.0, float(w_size))
    we2 = jnp.clip(jnp.ceil(cxb2 + hw2), 0.0, float(w_size))
    cnt = (he2 - hs2) * (we2 - ws2)
    sc_ref[...] = jnp.where(cnt > 0.0, 1.0 / jnp.maximum(cnt, 1.0), 0.0)


def _make_sc_gather(n_bins, n_rows):
    bins_per_worker = n_bins // NUM_WORKERS
    rois_per_worker = bins_per_worker // (P * P)
    n_chunks = rois_per_worker * 2
    mesh = plsc.VectorSubcoreMesh(core_axis_name="c", subcore_axis_name="s")

    @functools.partial(
        pl.kernel, mesh=mesh,
        out_type=jax.ShapeDtypeStruct((n_bins, 16), jnp.float32),
        scratch_types=[
            pltpu.VMEM((n_chunks, 128), jnp.int32),
            pltpu.VMEM((128, 16), jnp.float32),
            pltpu.VMEM((128, 16), jnp.float32),
            pltpu.VMEM((rois_per_worker * 56, 16), jnp.float32),
            pltpu.VMEM((bins_per_worker, 16), jnp.float32),
            pltpu.SemaphoreType.DMA,
            pltpu.SemaphoreType.DMA,
        ],
        compiler_params=pltpu.CompilerParams(use_tc_tiling_on_sc=False),
    )
    def sc_gather(idx_hbm, scale_hbm, table_hbm, out_hbm,
                  idx_v, rows_a, rows_b, scale_v, out_v, sem_a, sem_b):
        wid = lax.axis_index("s") * NUM_SC + lax.axis_index("c")
        pltpu.sync_copy(idx_hbm.at[pl.ds(wid * n_chunks, n_chunks)], idx_v)
        pltpu.sync_copy(
            scale_hbm.at[pl.ds(wid * rois_per_worker * 56,
                               rois_per_worker * 56)], scale_v)

        bufs = (rows_a, rows_b)
        sems = (sem_a, sem_b)

        def start(k):
            return pltpu.async_copy(table_hbm.at[idx_v.at[k]],
                                    bufs[k % 2], sems[k % 2])

        pending = start(0)
        for k in range(n_chunks):
            nxt = start(k + 1) if k + 1 < n_chunks else None
            pending.wait()
            rows = bufs[k % 2]
            r_local = k // 2
            nb = 32 if k % 2 == 0 else P * P - 32
            for i in range(nb):
                b_local = (k % 2) * 32 + i
                acc = (rows[4 * i, :] - rows[4 * i + 1, :]
                       - rows[4 * i + 2, :] + rows[4 * i + 3, :])
                out_v[r_local * (P * P) + b_local, :] = (
                    acc * scale_v[r_local * 56 + b_local, :])
            pending = nxt
        pltpu.sync_copy(
            out_v, out_hbm.at[pl.ds(wid * bins_per_worker, bins_per_worker)])

    return sc_gather


def kernel(rois, features, stride):
    B, C, H, W = features.shape
    D = C // (P * P)
    N = rois.shape[0]
    n_bins = N * P * P
    n_rows = B * P * P * (H + 1) * (W + 1)

    x_t = (features.reshape(B, D, P * P, H, W)
           .transpose(0, 2, 1, 3, 4)
           .reshape(B, P * P, D * H, W))

    u_mat = (np.arange(W)[:, None] < np.arange(W + 1)[None, :]).astype(np.float32)
    rr = np.arange(D * (H + 1))
    cc = np.arange(D * H)
    bd_mat = ((rr[:, None] // (H + 1) == cc[None, :] // H)
              & (cc[None, :] % H < rr[:, None] % (H + 1))).astype(np.float32)

    sscale = (1.0 / stride) * jnp.ones((1, 1), jnp.float32)
    nb = P
    body = functools.partial(_fused_tc_body, h_size=H, w_size=W, n_batch=nb)
    table_dmaj, idx4, scale16 = pl.pallas_call(
        body,
        grid=(B * P * P // nb,),
        in_specs=[
            pl.BlockSpec((1, nb, D * H, W),
                         lambda g, nb=nb, pp=P * P // nb: (g // pp, g % pp,
                                                           0, 0)),
            pl.BlockSpec((W, W + 1), lambda g: (0, 0)),
            pl.BlockSpec((D * (H + 1), D * H), lambda g: (0, 0)),
            pl.BlockSpec((N, 5), lambda g: (0, 0)),
            pl.BlockSpec((1, 1), lambda g: (0, 0)),
        ],
        out_specs=[
            pl.BlockSpec((nb, D * (H + 1), W + 1), lambda g: (g, 0, 0)),
            pl.BlockSpec((N, 256), lambda g: (0, 0)),
            pl.BlockSpec((N, 56 * 16), lambda g: (0, 0)),
        ],
        out_shape=[
            jax.ShapeDtypeStruct((B * P * P, D * (H + 1), W + 1), jnp.float32),
            jax.ShapeDtypeStruct((N, 256), jnp.int32),
            jax.ShapeDtypeStruct((N, 56 * 16), jnp.float32),
        ],
    )(x_t, jnp.asarray(u_mat, jnp.bfloat16), jnp.asarray(bd_mat, jnp.bfloat16),
      rois.astype(jnp.float32), sscale)

    table = table_dmaj.reshape(B * P * P, D, H + 1, W + 1).transpose(0, 2, 3, 1)
    table = jnp.pad(table, ((0, 0), (0, 0), (0, 0), (0, 16 - D)))
    table = table.reshape(n_rows, 16)

    idx_flat = idx4.reshape(N * 2, 128)
    scale16 = scale16.reshape(N * 56, 16)

    out_rows = _make_sc_gather(n_bins, n_rows)(idx_flat, scale16, table)

    return (out_rows[:, :D].reshape(N, P * P, D)
            .transpose(0, 2, 1).reshape(N, D, P, P))


# speedup vs baseline: 1.9498x; 1.0109x over previous
import functools

import numpy as np

import jax
import jax.numpy as jnp
from jax import lax
from jax.experimental import pallas as pl
from jax.experimental.pallas import tpu as pltpu
from jax.experimental.pallas import tpu_sc as plsc

P = 7
RSCALE = 1.2
BSCALE = 1.4
NUM_SC = 2
NUM_SUBCORES = 16
NUM_WORKERS = NUM_SC * NUM_SUBCORES


def _split_dot(m16, x, dn):
    hi = x.astype(jnp.bfloat16)
    lo = (x - hi.astype(jnp.float32)).astype(jnp.bfloat16)
    return (lax.dot_general(m16, hi, dn, preferred_element_type=jnp.float32)
            + lax.dot_general(m16, lo, dn, preferred_element_type=jnp.float32))


def _fused_tc_body(x_ref, u_ref, bd_ref, rois_ref, sscale_ref,
                   tab_ref, idx_ref, sc_ref, h_size, w_size, n_batch):
    dnl = (((1,), (0,)), ((), ()))
    for t in range(n_batch):
        a = _split_dot(bd_ref[...], x_ref[0, t], dnl)
        hi = a.astype(jnp.bfloat16)
        lo = (a - hi.astype(jnp.float32)).astype(jnp.bfloat16)
        tab_ref[t] = (
            lax.dot_general(hi, u_ref[...], dnl,
                            preferred_element_type=jnp.float32)
            + lax.dot_general(lo, u_ref[...], dnl,
                              preferred_element_type=jnp.float32))

    @pl.when(pl.program_id(0) == 0)
    def _():
        _bins_body(rois_ref, sscale_ref, idx_ref, sc_ref, h_size, w_size)


def _bins_body(rois_ref, sscale_ref, idx_ref, sc_ref, h_size, w_size):
    r = rois_ref[...]
    s = sscale_ref[0, 0]
    bidx = r[:, 0:1].astype(jnp.int32)
    x1 = r[:, 1:2] * s
    y1 = r[:, 2:3] * s
    x2 = r[:, 3:4] * s
    y2 = r[:, 4:5] * s
    roi_w = jnp.maximum(x2 - x1, 0.1)
    roi_h = jnp.maximum(y2 - y1, 0.1)
    cx = 0.5 * (x1 + x2)
    cy = 0.5 * (y1 + y2)
    w_s = roi_w * RSCALE
    h_s = roi_h * RSCALE
    x1s = cx - 0.5 * w_s
    y1s = cy - 0.5 * h_s
    bin_w = w_s / P
    bin_h = h_s / P

    col = lax.broadcasted_iota(jnp.int32, (1, 256), 1)
    pc = jnp.minimum(col // 4, P * P - 1)
    corner = col % 4
    ph = (pc // P).astype(jnp.float32)
    pw = (pc % P).astype(jnp.float32)
    cyb = y1s + (ph + 0.5) * bin_h
    cxb = x1s + (pw + 0.5) * bin_w
    half_h = 0.5 * BSCALE * bin_h
    half_w = 0.5 * BSCALE * bin_w
    hs = jnp.clip(jnp.floor(cyb - half_h), 0.0, float(h_size))
    he = jnp.clip(jnp.ceil(cyb + half_h), 0.0, float(h_size))
    ws = jnp.clip(jnp.floor(cxb - half_w), 0.0, float(w_size))
    we = jnp.clip(jnp.ceil(cxb + half_w), 0.0, float(w_size))
    ii = jnp.where(corner % 2 == 0, he, hs).astype(jnp.int32)
    jj = jnp.where(corner < 2, we, ws).astype(jnp.int32)
    wj = w_size + 1
    slab = (h_size + 1) * wj
    base = (bidx * (P * P) + pc) * slab
    idx_ref[...] = jnp.where(col < P * P * 4, base + ii * wj + jj, 0)

    col2 = lax.broadcasted_iota(jnp.int32, (1, 56 * 16), 1)
    pc2 = jnp.minimum(col2 // 16, P * P - 1)
    ph2 = (pc2 // P).astype(jnp.float32)
    pw2 = (pc2 % P).astype(jnp.float32)
    cyb2 = y1s + (ph2 + 0.5) * bin_h
    cxb2 = x1s + (pw2 + 0.5) * bin_w
    hh2 = 0.5 * BSCALE * bin_h
    hw2 = 0.5 * BSCALE * bin_w
    hs2 = jnp.clip(jnp.floor(cyb2 - hh2), 0.0, float(h_size))
    he2 = jnp.clip(jnp.ceil(cyb2 + hh2), 0.0, float(h_size))
    ws2 = jnp.clip(jnp.floor(cxb2 - hw2), 0.0, float(w_size))
    we2 = jnp.clip(jnp.ceil(cxb2 + hw2), 0.0, float(w_size))
    cnt = (he2 - hs2) * (we2 - ws2)
    sc_ref[...] = jnp.where(cnt > 0.0, 1.0 / jnp.maximum(cnt, 1.0), 0.0)


def _make_sc_gather(n_bins, n_rows):
    bins_per_worker = n_bins // NUM_WORKERS
    rois_per_worker = bins_per_worker // (P * P)
    n_chunks = rois_per_worker * 2
    mesh = plsc.VectorSubcoreMesh(core_axis_name="c", subcore_axis_name="s")

    @functools.partial(
        pl.kernel, mesh=mesh,
        out_type=jax.ShapeDtypeStruct((n_bins, 16), jnp.float32),
        scratch_types=[
            pltpu.VMEM((n_chunks, 128), jnp.int32),
            pltpu.VMEM((128, 16), jnp.float32),
            pltpu.VMEM((128, 16), jnp.float32),
            pltpu.VMEM((rois_per_worker * 56, 16), jnp.float32),
            pltpu.VMEM((bins_per_worker, 16), jnp.float32),
            pltpu.SemaphoreType.DMA,
            pltpu.SemaphoreType.DMA,
        ],
        compiler_params=pltpu.CompilerParams(use_tc_tiling_on_sc=False),
    )
    def sc_gather(idx_hbm, scale_hbm, table_hbm, out_hbm,
                  idx_v, rows_a, rows_b, scale_v, out_v, sem_a, sem_b):
        wid = lax.axis_index("s") * NUM_SC + lax.axis_index("c")
        pltpu.sync_copy(idx_hbm.at[pl.ds(wid * n_chunks, n_chunks)], idx_v)
        pltpu.sync_copy(
            scale_hbm.at[pl.ds(wid * rois_per_worker * 56,
                               rois_per_worker * 56)], scale_v)

        bufs = (rows_a, rows_b)
        sems = (sem_a, sem_b)
        n_pairs = rois_per_worker

        pltpu.async_copy(table_hbm.at[idx_v.at[0]], rows_a, sem_a)
        pltpu.async_copy(table_hbm.at[idx_v.at[1]], rows_b, sem_b)

        def pair_body(m, carry):
            for half in range(2):
                buf, sem = bufs[half], sems[half]
                pltpu.make_async_copy(table_hbm.at[pl.ds(0, 128)], buf,
                                      sem).wait()

                def bin_body(i, _, half=half, buf=buf):
                    acc = (buf[4 * i, :] - buf[4 * i + 1, :]
                           - buf[4 * i + 2, :] + buf[4 * i + 3, :])
                    out_v[m * (P * P) + half * 32 + i, :] = (
                        acc * scale_v[m * 56 + half * 32 + i, :])
                    return 0

                lax.fori_loop(0, 32 if half == 0 else P * P - 32,
                              bin_body, 0)

                @pl.when(m + 1 < n_pairs)
                def _(half=half, buf=buf, sem=sem):
                    pltpu.async_copy(
                        table_hbm.at[idx_v.at[2 * (m + 1) + half]], buf, sem)
            return carry

        lax.fori_loop(0, n_pairs, pair_body, 0)
        pltpu.sync_copy(
            out_v, out_hbm.at[pl.ds(wid * bins_per_worker, bins_per_worker)])

    return sc_gather


def kernel(rois, features, stride):
    B, C, H, W = features.shape
    D = C // (P * P)
    N = rois.shape[0]
    n_bins = N * P * P
    n_rows = B * P * P * (H + 1) * (W + 1)

    x_t = (features.reshape(B, D, P * P, H, W)
           .transpose(0, 2, 1, 3, 4)
           .reshape(B, P * P, D * H, W))

    u_mat = (np.arange(W)[:, None] < np.arange(W + 1)[None, :]).astype(np.float32)
    rr = np.arange(D * (H + 1))
    cc = np.arange(D * H)
    bd_mat = ((rr[:, None] // (H + 1) == cc[None, :] // H)
              & (cc[None, :] % H < rr[:, None] % (H + 1))).astype(np.float32)

    sscale = (1.0 / stride) * jnp.ones((1, 1), jnp.float32)
    nb = P
    body = functools.partial(_fused_tc_body, h_size=H, w_size=W, n_batch=nb)
    table_dmaj, idx4, scale16 = pl.pallas_call(
        body,
        grid=(B * P * P // nb,),
        in_specs=[
            pl.BlockSpec((1, nb, D * H, W),
                         lambda g, nb=nb, pp=P * P // nb: (g // pp, g % pp,
                                                           0, 0)),
            pl.BlockSpec((W, W + 1), lambda g: (0, 0)),
            pl.BlockSpec((D * (H + 1), D * H), lambda g: (0, 0)),
            pl.BlockSpec((N, 5), lambda g: (0, 0)),
            pl.BlockSpec((1, 1), lambda g: (0, 0)),
        ],
        out_specs=[
            pl.BlockSpec((nb, D * (H + 1), W + 1), lambda g: (g, 0, 0)),
            pl.BlockSpec((N, 256), lambda g: (0, 0)),
            pl.BlockSpec((N, 56 * 16), lambda g: (0, 0)),
        ],
        out_shape=[
            jax.ShapeDtypeStruct((B * P * P, D * (H + 1), W + 1), jnp.float32),
            jax.ShapeDtypeStruct((N, 256), jnp.int32),
            jax.ShapeDtypeStruct((N, 56 * 16), jnp.float32),
        ],
    )(x_t, jnp.asarray(u_mat, jnp.bfloat16), jnp.asarray(bd_mat, jnp.bfloat16),
      rois.astype(jnp.float32), sscale)

    table = table_dmaj.reshape(B * P * P, D, H + 1, W + 1).transpose(0, 2, 3, 1)
    table = jnp.pad(table, ((0, 0), (0, 0), (0, 0), (0, 16 - D)))
    table = table.reshape(n_rows, 16)

    idx_flat = idx4.reshape(N * 2, 128)
    scale16 = scale16.reshape(N * 56, 16)

    out_rows = _make_sc_gather(n_bins, n_rows)(idx_flat, scale16, table)

    return (out_rows[:, :D].reshape(N, P * P, D)
            .transpose(0, 2, 1).reshape(N, D, P, P))
